# baseline (speedup 1.0000x reference)
import functools

import jax
import jax.numpy as jnp
import numpy as np
from jax import lax
from jax.experimental import pallas as pl
from jax.experimental.pallas import tpu as pltpu
from jax.experimental.pallas import tpu_sc as plsc

B = 16384
V = 1000000
S = 50
D = 64
T = 1000

NC = 2
NS = 16
NW = NC * NS
BPW = B // NW
CB = 8
NCHUNK = BPW // CB
ROWS = CB * S

KV = 2048
NG = -(-V // KV)

_PERM = np.concatenate(
    [r * 32 + np.concatenate([np.arange(0, 32, 2), np.arange(1, 32, 2)])
     for r in range(D // 32)]
)


def _tc_repack(tableT):
    def body(x_ref, xn_ref, o_ref):
        x = x_ref[...]
        xn1 = xn_ref[...]
        xs = jnp.concatenate([x[:, 1:], xn1[:, :1]], axis=1)
        both = jnp.concatenate([x, xs], axis=0)
        o_ref[...] = lax.transpose(both, (1, 0)).astype(jnp.bfloat16)

    return pl.pallas_call(
        body,
        grid=(NG,),
        in_specs=[
            pl.BlockSpec((D, KV), lambda i: (0, i)),
            pl.BlockSpec(
                (D, 128),
                lambda i: (0, jnp.minimum((i + 1) * (KV // 128), V // 128 - 1)),
            ),
        ],
        out_specs=pl.BlockSpec((KV, 2 * D), lambda i: (i, 0)),
        out_shape=jax.ShapeDtypeStruct((V, 2 * D), jnp.bfloat16),
    )(tableT, tableT)


def _sc_embed_sum(pieces_flat, table):
    mesh = plsc.VectorSubcoreMesh(core_axis_name="c", subcore_axis_name="s")

    @functools.partial(
        pl.kernel,
        mesh=mesh,
        out_type=jax.ShapeDtypeStruct((B, D), jnp.float32),
        compiler_params=pltpu.CompilerParams(use_tc_tiling_on_sc=False, needs_layout_passes=False),
        scratch_types=[
            pltpu.VMEM((BPW * S,), jnp.int32),
            pltpu.VMEM((ROWS, 2 * D), jnp.bfloat16),
            pltpu.VMEM((ROWS, 2 * D), jnp.bfloat16),
            pltpu.VMEM((BPW, D), jnp.float32),
            pltpu.SemaphoreType.DMA,
            pltpu.SemaphoreType.DMA,
        ],
    )
    def k(pieces_hbm, table_hbm, out_hbm, idx_v, rows0, rows1, feat_v, sem0, sem1):
        wid = lax.axis_index("s") * NC + lax.axis_index("c")
        base = wid * (BPW * S)
        pltpu.sync_copy(pieces_hbm.at[pl.ds(base, BPW * S)], idx_v)
        rows = (rows0, rows1)
        sems = (sem0, sem1)

        def gather(chunk, p):
            off = pl.multiple_of(chunk * ROWS, 8)
            return pltpu.make_async_copy(
                table_hbm.at[idx_v.at[pl.ds(off, ROWS)]], rows[p], sems[p]
            )

        for p in range(2):
            gather(p, p).start()

        mask = jnp.full((16,), 0xFFFF0000, dtype=jnp.uint32)

        def load_row(buf, r):
            out = []
            for h in range(2):
                u = plsc.bitcast(buf[r, pl.ds(h * 32, 32)], jnp.uint32)
                out.append(plsc.bitcast(u << 16, jnp.float32))
                out.append(plsc.bitcast(u & mask, jnp.float32))
            return out

        def accumulate(buf, chunk):
            def per_b(bi, _):
                r0 = bi * S
                acc = load_row(buf, r0)

                def per_s(si, acc):
                    rr = r0 + 1 + si * 7
                    for j in range(7):
                        row = load_row(buf, rr + j)
                        for r in range(4):
                            acc[r] = acc[r] + row[r]
                    return acc

                acc = lax.fori_loop(0, 7, per_s, acc)
                row = chunk * CB + bi
                for r in range(4):
                    feat_v[row, pl.ds(r * 16, 16)] = acc[r]
                return 0

            lax.fori_loop(0, CB, per_b, 0)

        def outer(i, _):
            g = i * 2
            for p in range(2):
                chunk = g + p
                gather(chunk, p).wait()

                @pl.when(chunk + 2 < NCHUNK)
                def _():
                    gather(chunk + 2, p).start()

                accumulate(rows[p], chunk)
            return 0

        lax.fori_loop(0, NCHUNK // 2, outer, 0)
        out_off = pl.multiple_of(wid * BPW, 8)
        pltpu.sync_copy(feat_v, out_hbm.at[pl.ds(out_off, BPW)])

    return k(pieces_flat, table)


def _tc_head(feature, Wp, b2):
    BB = 2048

    def body(x_ref, w_ref, b_ref, o_ref):
        z = lax.dot_general(
            w_ref[...], x_ref[...], (((1,), (1,)), ((), ())),
            preferred_element_type=jnp.float32,
        )
        z = z + b_ref[...]
        o_ref[...] = 1.0 / (1.0 + jnp.exp(-z))

    return pl.pallas_call(
        body,
        grid=(B // BB,),
        in_specs=[
            pl.BlockSpec((BB, D), lambda i: (i, 0)),
            pl.BlockSpec((T, D), lambda i: (0, 0)),
            pl.BlockSpec((T, 1), lambda i: (0, 0)),
        ],
        out_specs=pl.BlockSpec((T, BB), lambda i: (0, i)),
        out_shape=jax.ShapeDtypeStruct((T, B), jnp.float32),
    )(feature, Wp, b2)


def kernel(pieces, emb_table, W, b):
    table_pairs = _tc_repack(emb_table.T)
    feature = _sc_embed_sum(pieces.reshape(-1), table_pairs)
    Wp = W[:, _PERM]
    return _tc_head(feature, Wp, b.reshape(T, 1)).T


# speedup vs baseline: 3.6463x; 3.6463x over previous
import functools

import jax
import jax.numpy as jnp
import numpy as np
from jax import lax
from jax.experimental import pallas as pl
from jax.experimental.pallas import tpu as pltpu
from jax.experimental.pallas import tpu_sc as plsc

B = 16384
V = 1000000
S = 50
D = 64
T = 1000

NC = 2
NS = 16
NW = NC * NS
BPW = B // NW
CB = 8
NCHUNK = BPW // CB
ROWS = CB * S

KV = 2048
NBLK = -(-V // KV)
NQ = -(-NBLK // 4)
XROWS = NQ * KV
TROWS = 4 * XROWS

_PERM = np.concatenate([
    np.arange(32, 48), np.arange(0, 16),
    np.arange(48, 64), np.arange(16, 32),
])


def _tc_repack(tableT):
    def body(x0_ref, x1_ref, x2_ref, x3_ref, o_ref):
        cols = []
        for xr in (x0_ref, x1_ref, x2_ref, x3_ref):
            x = xr[...]
            ta = x[0:32, :].astype(jnp.bfloat16).astype(jnp.float32)
            tb = x[32:64, :].astype(jnp.bfloat16).astype(jnp.float32)
            ia = lax.bitcast_convert_type(ta, jnp.int32)
            ib = lax.bitcast_convert_type(tb, jnp.int32)
            w = ia | lax.shift_right_logical(ib, 16)
            wf = lax.bitcast_convert_type(w, jnp.float32)
            wt = lax.transpose(wf, (1, 0))
            cols.append(lax.bitcast_convert_type(wt, jnp.int32))
        o_ref[...] = jnp.concatenate(cols, axis=1)

    def spec(g):
        return pl.BlockSpec(
            (D, KV), lambda i, g=g: (0, jnp.minimum(4 * i + g, NBLK - 1)))

    return pl.pallas_call(
        body,
        grid=(NQ,),
        in_specs=[spec(0), spec(1), spec(2), spec(3)],
        out_specs=pl.BlockSpec((KV, 128), lambda i: (i, 0)),
        out_shape=jax.ShapeDtypeStruct((XROWS, 128), jnp.int32),
    )(tableT, tableT, tableT, tableT)


def _sc_embed_sum(idx_flat, table):
    mesh = plsc.VectorSubcoreMesh(core_axis_name="c", subcore_axis_name="s")

    @functools.partial(
        pl.kernel,
        mesh=mesh,
        out_type=jax.ShapeDtypeStruct((B, D), jnp.float32),
        compiler_params=pltpu.CompilerParams(
            use_tc_tiling_on_sc=False, needs_layout_passes=False),
        scratch_types=[
            pltpu.VMEM((BPW * S,), jnp.int32),
            pltpu.VMEM((ROWS, 32), jnp.int32),
            pltpu.VMEM((ROWS, 32), jnp.int32),
            pltpu.VMEM((BPW, D), jnp.float32),
            pltpu.SemaphoreType.DMA,
            pltpu.SemaphoreType.DMA,
        ],
    )
    def k(idx_hbm, table_hbm, out_hbm, idx_v, rows0, rows1, feat_v, sem0, sem1):
        wid = lax.axis_index("s") * NC + lax.axis_index("c")
        base = wid * (BPW * S)
        pltpu.sync_copy(idx_hbm.at[pl.ds(base, BPW * S)], idx_v)
        rows = (rows0, rows1)
        sems = (sem0, sem1)

        def gather(chunk, p):
            off = pl.multiple_of(chunk * ROWS, 8)
            return pltpu.make_async_copy(
                table_hbm.at[idx_v.at[pl.ds(off, ROWS)]], rows[p], sems[p]
            )

        for p in range(2):
            gather(p, p).start()

        mask = jnp.full((16,), -65536, dtype=jnp.int32)

        def load_row(buf, r):
            out = []
            for h in range(2):
                u = buf[r, pl.ds(h * 16, 16)]
                out.append(plsc.bitcast(u << 16, jnp.float32))
                out.append(plsc.bitcast(u & mask, jnp.float32))
            return out

        def accumulate(buf, chunk):
            def per_b(bi, _):
                r0 = bi * S
                acc = load_row(buf, r0)

                def per_s(si, acc):
                    rr = r0 + 1 + si * 7
                    for j in range(7):
                        row = load_row(buf, rr + j)
                        for r in range(4):
                            acc[r] = acc[r] + row[r]
                    return acc

                acc = lax.fori_loop(0, 7, per_s, acc)
                row = chunk * CB + bi
                for r in range(4):
                    feat_v[row, pl.ds(r * 16, 16)] = acc[r]
                return 0

            lax.fori_loop(0, CB, per_b, 0)

        def outer(i, _):
            g = i * 2
            for p in range(2):
                chunk = g + p
                gather(chunk, p).wait()

                @pl.when(chunk + 2 < NCHUNK)
                def _():
                    gather(chunk + 2, p).start()

                accumulate(rows[p], chunk)
            return 0

        lax.fori_loop(0, NCHUNK // 2, outer, 0)
        out_off = pl.multiple_of(wid * BPW, 8)
        pltpu.sync_copy(feat_v, out_hbm.at[pl.ds(out_off, BPW)])

    return k(idx_flat, table)


def _tc_head(feature, Wp, b2):
    BB = 2048

    def body(x_ref, w_ref, b_ref, o_ref):
        z = lax.dot_general(
            w_ref[...], x_ref[...], (((1,), (1,)), ((), ())),
            preferred_element_type=jnp.float32,
        )
        z = z + b_ref[...]
        o_ref[...] = 1.0 / (1.0 + jnp.exp(-z))

    return pl.pallas_call(
        body,
        grid=(B // BB,),
        in_specs=[
            pl.BlockSpec((BB, D), lambda i: (i, 0)),
            pl.BlockSpec((T, D), lambda i: (0, 0)),
            pl.BlockSpec((T, 1), lambda i: (0, 0)),
        ],
        out_specs=pl.BlockSpec((T, BB), lambda i: (0, i)),
        out_shape=jax.ShapeDtypeStruct((T, B), jnp.float32),
    )(feature, Wp, b2)


def kernel(pieces, emb_table, W, b):
    table = _tc_repack(emb_table.T).reshape(TROWS, 32)
    v = pieces.reshape(-1)
    pos = v % (4 * KV)
    m = 4 * ((v - pos) // 4 + (pos % KV)) + pos // KV
    feature = _sc_embed_sum(m, table)
    Wp = W[:, _PERM]
    return _tc_head(feature, Wp, b.reshape(T, 1)).T


# speedup vs baseline: 3.7457x; 1.0273x over previous
import functools

import jax
import jax.numpy as jnp
import numpy as np
from jax import lax
from jax.experimental import pallas as pl
from jax.experimental.pallas import tpu as pltpu
from jax.experimental.pallas import tpu_sc as plsc

B = 16384
V = 1000000
S = 50
D = 64
T = 1000

NC = 2
NS = 16
NW = NC * NS
BPW = B // NW
CB = 8
NCHUNK = BPW // CB
ROWS = CB * S

KV = 4096
NBLK = -(-V // KV)
NQ = -(-NBLK // 4)
XROWS = NQ * KV
TROWS = 4 * XROWS

_PERM = np.concatenate([
    np.arange(32, 48), np.arange(0, 16),
    np.arange(48, 64), np.arange(16, 32),
])


def _tc_repack(tableT):
    def body(x0_ref, x1_ref, x2_ref, x3_ref, o_ref):
        cols = []
        for xr in (x0_ref, x1_ref, x2_ref, x3_ref):
            x = xr[...]
            ta = x[0:32, :].astype(jnp.bfloat16).astype(jnp.float32)
            tb = x[32:64, :].astype(jnp.bfloat16).astype(jnp.float32)
            ia = lax.bitcast_convert_type(ta, jnp.int32)
            ib = lax.bitcast_convert_type(tb, jnp.int32)
            w = ia | lax.shift_right_logical(ib, 16)
            wf = lax.bitcast_convert_type(w, jnp.float32)
            wt = lax.transpose(wf, (1, 0))
            cols.append(lax.bitcast_convert_type(wt, jnp.int32))
        o_ref[...] = jnp.concatenate(cols, axis=1)

    def spec(g):
        return pl.BlockSpec(
            (D, KV), lambda i, g=g: (0, jnp.minimum(4 * i + g, NBLK - 1)))

    return pl.pallas_call(
        body,
        grid=(NQ,),
        in_specs=[spec(0), spec(1), spec(2), spec(3)],
        out_specs=pl.BlockSpec((KV, 128), lambda i: (i, 0)),
        out_shape=jax.ShapeDtypeStruct((XROWS, 128), jnp.int32),
    )(tableT, tableT, tableT, tableT)


def _sc_embed_sum(idx_flat, table):
    mesh = plsc.VectorSubcoreMesh(core_axis_name="c", subcore_axis_name="s")

    @functools.partial(
        pl.kernel,
        mesh=mesh,
        out_type=jax.ShapeDtypeStruct((B, D), jnp.float32),
        compiler_params=pltpu.CompilerParams(
            use_tc_tiling_on_sc=False, needs_layout_passes=False),
        scratch_types=[
            pltpu.VMEM((BPW * S,), jnp.int32),
            pltpu.VMEM((ROWS, 32), jnp.int32),
            pltpu.VMEM((ROWS, 32), jnp.int32),
            pltpu.VMEM((BPW, D), jnp.float32),
            pltpu.SemaphoreType.DMA,
            pltpu.SemaphoreType.DMA,
        ],
    )
    def k(idx_hbm, table_hbm, out_hbm, idx_v, rows0, rows1, feat_v, sem0, sem1):
        wid = lax.axis_index("s") * NC + lax.axis_index("c")
        base = wid * (BPW * S)
        pltpu.sync_copy(idx_hbm.at[pl.ds(base, BPW * S)], idx_v)
        rows = (rows0, rows1)
        sems = (sem0, sem1)

        def gather(chunk, p):
            off = pl.multiple_of(chunk * ROWS, 8)
            return pltpu.make_async_copy(
                table_hbm.at[idx_v.at[pl.ds(off, ROWS)]], rows[p], sems[p]
            )

        for p in range(2):
            gather(p, p).start()

        mask = jnp.full((16,), -65536, dtype=jnp.int32)

        def load_row(buf, r):
            out = []
            for h in range(2):
                u = buf[r, pl.ds(h * 16, 16)]
                out.append(plsc.bitcast(u << 16, jnp.float32))
                out.append(plsc.bitcast(u & mask, jnp.float32))
            return out

        def accumulate(buf, chunk):
            def per_b(bi, _):
                r0 = bi * S
                acc = load_row(buf, r0)

                def per_s(si, acc):
                    rr = r0 + 1 + si * 7
                    for j in range(7):
                        row = load_row(buf, rr + j)
                        for r in range(4):
                            acc[r] = acc[r] + row[r]
                    return acc

                acc = lax.fori_loop(0, 7, per_s, acc)
                row = chunk * CB + bi
                for r in range(4):
                    feat_v[row, pl.ds(r * 16, 16)] = acc[r]
                return 0

            lax.fori_loop(0, CB, per_b, 0)

        def outer(i, _):
            g = i * 2
            for p in range(2):
                chunk = g + p
                gather(chunk, p).wait()

                @pl.when(chunk + 2 < NCHUNK)
                def _():
                    gather(chunk + 2, p).start()

                accumulate(rows[p], chunk)
            return 0

        lax.fori_loop(0, NCHUNK // 2, outer, 0)
        out_off = pl.multiple_of(wid * BPW, 8)
        pltpu.sync_copy(feat_v, out_hbm.at[pl.ds(out_off, BPW)])

    return k(idx_flat, table)


def _tc_head(feature, Wp, b2):
    BB = 2048

    def body(x_ref, w_ref, b_ref, o_ref):
        z = lax.dot_general(
            w_ref[...], x_ref[...], (((1,), (1,)), ((), ())),
            preferred_element_type=jnp.float32,
        )
        z = z + b_ref[...]
        o_ref[...] = 1.0 / (1.0 + jnp.exp(-z))

    return pl.pallas_call(
        body,
        grid=(B // BB,),
        in_specs=[
            pl.BlockSpec((BB, D), lambda i: (i, 0)),
            pl.BlockSpec((T, D), lambda i: (0, 0)),
            pl.BlockSpec((T, 1), lambda i: (0, 0)),
        ],
        out_specs=pl.BlockSpec((T, BB), lambda i: (0, i)),
        out_shape=jax.ShapeDtypeStruct((T, B), jnp.float32),
    )(feature, Wp, b2)


def kernel(pieces, emb_table, W, b):
    table = _tc_repack(emb_table.T).reshape(TROWS, 32)
    v = pieces.reshape(-1)
    pos = v % (4 * KV)
    m = 4 * ((v - pos) // 4 + (pos % KV)) + pos // KV
    feature = _sc_embed_sum(m, table)
    Wp = W[:, _PERM]
    return _tc_head(feature, Wp, b.reshape(T, 1)).T


# speedup vs baseline: 3.7855x; 1.0106x over previous
import functools

import jax
import jax.numpy as jnp
import numpy as np
from jax import lax
from jax.experimental import pallas as pl
from jax.experimental.pallas import tpu as pltpu
from jax.experimental.pallas import tpu_sc as plsc

B = 16384
V = 1000000
S = 50
D = 64
T = 1000

NC = 2
NS = 16
NW = NC * NS
BPW = B // NW
CB = 8
NCHUNK = BPW // CB
ROWS = CB * S

KV = 8192
NBLK = -(-V // KV)
NQ = -(-NBLK // 4)
XROWS = NQ * KV
TROWS = 4 * XROWS

_PERM = np.concatenate([
    np.arange(32, 48), np.arange(0, 16),
    np.arange(48, 64), np.arange(16, 32),
])


def _tc_repack(tableT):
    def body(x0_ref, x1_ref, x2_ref, x3_ref, o_ref):
        cols = []
        for xr in (x0_ref, x1_ref, x2_ref, x3_ref):
            x = xr[...]
            ta = x[0:32, :].astype(jnp.bfloat16).astype(jnp.float32)
            tb = x[32:64, :].astype(jnp.bfloat16).astype(jnp.float32)
            ia = lax.bitcast_convert_type(ta, jnp.int32)
            ib = lax.bitcast_convert_type(tb, jnp.int32)
            w = ia | lax.shift_right_logical(ib, 16)
            wf = lax.bitcast_convert_type(w, jnp.float32)
            wt = lax.transpose(wf, (1, 0))
            cols.append(lax.bitcast_convert_type(wt, jnp.int32))
        o_ref[...] = jnp.concatenate(cols, axis=1)

    def spec(g):
        return pl.BlockSpec(
            (D, KV), lambda i, g=g: (0, jnp.minimum(4 * i + g, NBLK - 1)))

    return pl.pallas_call(
        body,
        grid=(NQ,),
        in_specs=[spec(0), spec(1), spec(2), spec(3)],
        out_specs=pl.BlockSpec((KV, 128), lambda i: (i, 0)),
        out_shape=jax.ShapeDtypeStruct((XROWS, 128), jnp.int32),
    )(tableT, tableT, tableT, tableT)


def _sc_embed_sum(idx_flat, table):
    mesh = plsc.VectorSubcoreMesh(core_axis_name="c", subcore_axis_name="s")

    @functools.partial(
        pl.kernel,
        mesh=mesh,
        out_type=jax.ShapeDtypeStruct((B, D), jnp.float32),
        compiler_params=pltpu.CompilerParams(
            use_tc_tiling_on_sc=False, needs_layout_passes=False),
        scratch_types=[
            pltpu.VMEM((BPW * S,), jnp.int32),
            pltpu.VMEM((ROWS, 32), jnp.int32),
            pltpu.VMEM((ROWS, 32), jnp.int32),
            pltpu.VMEM((BPW, D), jnp.float32),
            pltpu.SemaphoreType.DMA,
            pltpu.SemaphoreType.DMA,
        ],
    )
    def k(idx_hbm, table_hbm, out_hbm, idx_v, rows0, rows1, feat_v, sem0, sem1):
        wid = lax.axis_index("s") * NC + lax.axis_index("c")
        base = wid * (BPW * S)
        pltpu.sync_copy(idx_hbm.at[pl.ds(base, BPW * S)], idx_v)
        rows = (rows0, rows1)
        sems = (sem0, sem1)

        def gather(chunk, p):
            off = pl.multiple_of(chunk * ROWS, 8)
            return pltpu.make_async_copy(
                table_hbm.at[idx_v.at[pl.ds(off, ROWS)]], rows[p], sems[p]
            )

        for p in range(2):
            gather(p, p).start()

        mask = jnp.full((16,), -65536, dtype=jnp.int32)

        def load_row(buf, r):
            out = []
            for h in range(2):
                u = buf[r, pl.ds(h * 16, 16)]
                out.append(plsc.bitcast(u << 16, jnp.float32))
                out.append(plsc.bitcast(u & mask, jnp.float32))
            return out

        def accumulate(buf, chunk):
            def per_b(bi, _):
                r0 = bi * S
                acc = load_row(buf, r0)

                def per_s(si, acc):
                    rr = r0 + 1 + si * 7
                    for j in range(7):
                        row = load_row(buf, rr + j)
                        for r in range(4):
                            acc[r] = acc[r] + row[r]
                    return acc

                acc = lax.fori_loop(0, 7, per_s, acc)
                row = chunk * CB + bi
                for r in range(4):
                    feat_v[row, pl.ds(r * 16, 16)] = acc[r]
                return 0

            lax.fori_loop(0, CB, per_b, 0)

        def outer(i, _):
            g = i * 2
            for p in range(2):
                chunk = g + p
                gather(chunk, p).wait()

                @pl.when(chunk + 2 < NCHUNK)
                def _():
                    gather(chunk + 2, p).start()

                accumulate(rows[p], chunk)
            return 0

        lax.fori_loop(0, NCHUNK // 2, outer, 0)
        out_off = pl.multiple_of(wid * BPW, 8)
        pltpu.sync_copy(feat_v, out_hbm.at[pl.ds(out_off, BPW)])

    return k(idx_flat, table)


def _tc_head(feature, Wp, b2):
    BB = 2048

    def body(x_ref, w_ref, b_ref, o_ref):
        z = lax.dot_general(
            w_ref[...], x_ref[...], (((1,), (1,)), ((), ())),
            preferred_element_type=jnp.float32,
        )
        z = z + b_ref[...]
        o_ref[...] = 1.0 / (1.0 + jnp.exp(-z))

    return pl.pallas_call(
        body,
        grid=(B // BB,),
        in_specs=[
            pl.BlockSpec((BB, D), lambda i: (i, 0)),
            pl.BlockSpec((T, D), lambda i: (0, 0)),
            pl.BlockSpec((T, 1), lambda i: (0, 0)),
        ],
        out_specs=pl.BlockSpec((T, BB), lambda i: (0, i)),
        out_shape=jax.ShapeDtypeStruct((T, B), jnp.float32),
    )(feature, Wp, b2)


def kernel(pieces, emb_table, W, b):
    table = _tc_repack(emb_table.T).reshape(TROWS, 32)
    v = pieces.reshape(-1)
    pos = v % (4 * KV)
    m = 4 * ((v - pos) // 4 + (pos % KV)) + pos // KV
    feature = _sc_embed_sum(m, table)
    Wp = W[:, _PERM]
    return _tc_head(feature, Wp, b.reshape(T, 1)).T
